# baseline (speedup 1.0000x reference)
import functools

import jax
import jax.numpy as jnp
from jax import lax
from jax.experimental import pallas as pl
from jax.experimental.pallas import tpu as pltpu
from jax.experimental.pallas import tpu_sc as plsc

B = 4096
L = 50
NP = B // 2
D = 300
V = 517015
VB = 253
V2 = VB * 2048
NW = 32
PPW = NP // NW
GP = 104
INV_L = 1.0 / L

_mesh = plsc.VectorSubcoreMesh(core_axis_name="c", subcore_axis_name="s")


def _retile_body(tt_ref, s_ref):
    s_ref[...] = jnp.transpose(tt_ref[...], (1, 0))


def _retile_piece(tt, g):
    return pl.pallas_call(
        _retile_body,
        grid=(VB,),
        in_specs=[pl.BlockSpec((128, 2048), lambda j, g=g: (g, j))],
        out_specs=pl.BlockSpec((2048, 128), lambda j: (j, 0)),
        out_shape=jax.ShapeDtypeStruct((V2, 128), jnp.float32),
    )(tt)


def _reduce_pair(rows, out_v, w0):
    for half in (0, 1):
        def word_body(j, accs):
            return tuple(accs[u] + rows[j, pl.ds(u * 16, 16)]
                         for u in range(8))

        init = tuple(jnp.zeros((16,), jnp.float32) for _ in range(8))
        accs = lax.fori_loop(half * L, (half + 1) * L, word_body, init,
                             unroll=5)
        for u in range(8):
            out_v[w0 + half, pl.ds(u * 16, 16)] = accs[u] * INV_L


@functools.partial(
    pl.kernel,
    out_type=jax.ShapeDtypeStruct((B, 128), jnp.float32),
    mesh=_mesh,
    scratch_types=[
        pltpu.VMEM((PPW, GP), jnp.int32),
        pltpu.VMEM((GP, 128), jnp.float32),
        pltpu.VMEM((GP, 128), jnp.float32),
        pltpu.VMEM((2 * PPW, 128), jnp.float32),
        pltpu.SemaphoreType.DMA,
        pltpu.SemaphoreType.DMA,
    ],
    compiler_params=pltpu.CompilerParams(use_tc_tiling_on_sc=False),
)
def _pool_piece(idx_hbm, s_hbm, out_hbm, idx_v, rows0, rows1,
                out_v, sem0, sem1):
    wid = lax.axis_index("s") * 2 + lax.axis_index("c")
    base = wid * PPW

    pltpu.sync_copy(idx_hbm.at[pl.ds(base, PPW)], idx_v)
    pltpu.async_copy(s_hbm.at[idx_v.at[0]], rows0, sem0)

    def pair_body(i, _):
        p0 = 2 * i
        cp1 = pltpu.async_copy(s_hbm.at[idx_v.at[p0 + 1]], rows1, sem1)
        pltpu.make_async_copy(s_hbm.at[idx_v.at[0]], rows0, sem0).wait()
        _reduce_pair(rows0, out_v, 2 * p0)

        @pl.when(i < PPW // 2 - 1)
        def _():
            pltpu.async_copy(s_hbm.at[idx_v.at[p0 + 2]], rows0, sem0)

        cp1.wait()
        _reduce_pair(rows1, out_v, 2 * p0 + 2)
        return 0

    lax.fori_loop(0, PPW // 2, pair_body, 0)

    pltpu.sync_copy(out_v, out_hbm.at[pl.ds(2 * base, 2 * PPW)])


def kernel(indices, table):
    idx = jnp.pad(indices.astype(jnp.int32).reshape(NP, 2 * L),
                  ((0, 0), (0, GP - 2 * L)))
    tt = table.T
    outs = []
    for g in range(3):
        s = _retile_piece(tt, g)
        outs.append(_pool_piece(idx, s))
    return jnp.concatenate(outs, axis=1)[:, :D]


# speedup vs baseline: 1.4116x; 1.4116x over previous
import functools

import jax
import jax.numpy as jnp
from jax import lax
from jax.experimental import pallas as pl
from jax.experimental.pallas import tpu as pltpu
from jax.experimental.pallas import tpu_sc as plsc

B = 4096
L = 50
D = 300
V = 517015
VB = 253
V2 = VB * 2048
NW = 32
SPW = B // NW
GP = 152
INV_L = 1.0 / L

_mesh = plsc.VectorSubcoreMesh(core_axis_name="c", subcore_axis_name="s")


def _retile_body(tt_ref, s_ref):
    s_ref[...] = jnp.transpose(tt_ref[...], (1, 0))


@jax.jit
def _retile(tt):
    return pl.pallas_call(
        _retile_body,
        grid=(3, VB),
        in_specs=[pl.BlockSpec((128, 2048), lambda g, j: (g, j))],
        out_specs=pl.BlockSpec((2048, 128), lambda g, j: (g * VB + j, 0)),
        out_shape=jax.ShapeDtypeStruct((3 * V2, 128), jnp.float32),
    )(tt)


def _reduce_sent(rows, out_v, w):
    def word_body(j, accs):
        r0 = 3 * j
        a = [accs[u] + rows[r0, pl.ds(u * 16, 16)] for u in range(8)]
        b = [accs[8 + u] + rows[r0 + 1, pl.ds(u * 16, 16)] for u in range(8)]
        c = [accs[16 + u] + rows[r0 + 2, pl.ds(u * 16, 16)] for u in range(3)]
        return tuple(a + b + c)

    init = tuple(jnp.zeros((16,), jnp.float32) for _ in range(19))
    accs = lax.fori_loop(0, L, word_body, init, unroll=5)
    for u in range(8):
        out_v[w, pl.ds(u * 16, 16)] = accs[u] * INV_L
    for u in range(8):
        out_v[w, pl.ds(128 + u * 16, 16)] = accs[8 + u] * INV_L
    for u in range(3):
        out_v[w, pl.ds(256 + u * 16, 16)] = accs[16 + u] * INV_L


@functools.partial(
    pl.kernel,
    out_type=jax.ShapeDtypeStruct((B, 304), jnp.float32),
    mesh=_mesh,
    scratch_types=[
        pltpu.VMEM((SPW, GP), jnp.int32),
        pltpu.VMEM((GP, 128), jnp.float32),
        pltpu.VMEM((GP, 128), jnp.float32),
        pltpu.VMEM((SPW, 304), jnp.float32),
        pltpu.SemaphoreType.DMA,
        pltpu.SemaphoreType.DMA,
    ],
    compiler_params=pltpu.CompilerParams(use_tc_tiling_on_sc=False),
)
def _pooled_lookup(idx_hbm, s_hbm, out_hbm, idx_v, rows0, rows1,
                   out_v, sem0, sem1):
    wid = lax.axis_index("s") * 2 + lax.axis_index("c")
    base = wid * SPW

    pltpu.sync_copy(idx_hbm.at[pl.ds(base, SPW)], idx_v)
    pltpu.async_copy(s_hbm.at[idx_v.at[0]], rows0, sem0)

    def sent_body(i, _):
        w0 = 2 * i
        cp1 = pltpu.async_copy(s_hbm.at[idx_v.at[w0 + 1]], rows1, sem1)
        pltpu.make_async_copy(s_hbm.at[idx_v.at[0]], rows0, sem0).wait()
        _reduce_sent(rows0, out_v, w0)

        @pl.when(i < SPW // 2 - 1)
        def _():
            pltpu.async_copy(s_hbm.at[idx_v.at[w0 + 2]], rows0, sem0)

        cp1.wait()
        _reduce_sent(rows1, out_v, w0 + 1)
        return 0

    lax.fori_loop(0, SPW // 2, sent_body, 0)

    pltpu.sync_copy(out_v, out_hbm.at[pl.ds(base, SPW)])


def kernel(indices, table):
    idx = indices.astype(jnp.int32)
    piece = jnp.array([0, V2, 2 * V2], dtype=jnp.int32)
    idx3 = (idx[:, :, None] + piece[None, None, :]).reshape(B, 3 * L)
    idx3 = jnp.pad(idx3, ((0, 0), (0, GP - 3 * L)))
    s = _retile(table.T)
    return _pooled_lookup(idx3, s)[:, :D]
